# baseline (speedup 1.0000x reference)
import functools

import jax
import jax.numpy as jnp
from jax import lax
from jax.experimental import pallas as pl
from jax.experimental.pallas import tpu as pltpu


def _build_patches(xr, w_img, hw):
    m = lax.broadcasted_iota(jnp.int32, (1, hw), 1)
    w_pos = m % w_img
    col_ok = {
        0: w_pos >= 1,
        1: None,
        2: w_pos <= w_img - 2,
    }
    row_ok = {
        0: m >= w_img,
        1: None,
        2: m < hw - w_img,
    }
    taps = []
    for kh in range(3):
        for kw in range(3):
            d = (kh - 1) * w_img + (kw - 1)
            sh = xr if d == 0 else pltpu.roll(xr, (-d) % hw, axis=1)
            mask = row_ok[kh]
            if col_ok[kw] is not None:
                mask = col_ok[kw] if mask is None else (mask & col_ok[kw])
            if mask is not None:
                sh = jnp.where(mask, sh, 0.0)
            taps.append(sh)
    return jnp.concatenate(taps, axis=0)


def _stats_kernel(x_ref, g_ref, s_ref, *, w_img, hw):
    t = pl.program_id(1)

    @pl.when(t == 0)
    def _():
        g_ref[...] = jnp.zeros_like(g_ref)
        s_ref[...] = jnp.zeros_like(s_ref)

    p = _build_patches(x_ref[0], w_img, hw)
    g_ref[0] += lax.dot_general(p, p, (((1,), (1,)), ((), ())),
                                preferred_element_type=jnp.float32)
    s_ref[0] += jnp.sum(p, axis=1, keepdims=True)


def _conv_bn_relu_kernel(x_ref, wm_ref, wp_ref, g_ref, s_ref, gam_ref,
                         bet_ref, o_ref, *, w_img, hw, pack, inv_m, eps):
    wm = wm_ref[...]
    mean = jnp.dot(wm, s_ref[...],
                   preferred_element_type=jnp.float32) * inv_m
    wg = jnp.dot(wm, g_ref[...], preferred_element_type=jnp.float32)
    sq = jnp.sum(wg * wm, axis=1, keepdims=True)
    var = sq * inv_m - mean * mean
    scale = gam_ref[...] * lax.rsqrt(var + eps)
    shift = bet_ref[...] - mean * scale

    p = _build_patches(x_ref[0], w_img, hw)
    C = wm.shape[0]
    h_img = hw // w_img
    for i in range(pack):
        y = jnp.dot(wp_ref[i], p, preferred_element_type=jnp.float32)
        y = jnp.maximum(y * scale + shift, 0.0)
        o_ref[i] = y.reshape(C, h_img, w_img)


def kernel(x_nchw, w_hwio, bias, gamma, beta, *, eps=1e-5):
    N, cin, h_img, w_img = x_nchw.shape
    assert cin == 3
    C = w_hwio.shape[-1]
    hw = h_img * w_img
    M = N * hw
    del bias

    pack = 2 if N % 2 == 0 else 1
    groups = N // pack
    rows = cin * pack
    K = 9 * rows

    xpack = x_nchw.reshape(groups, rows, hw)
    w_mat = w_hwio.reshape(9 * cin, C).T
    wk = w_mat.reshape(C, 9, 1, cin)
    slots = []
    for i in range(pack):
        zpre = jnp.zeros((C, 9, 1, cin * i), jnp.float32)
        zpost = jnp.zeros((C, 9, 1, cin * (pack - 1 - i)), jnp.float32)
        slots.append(jnp.concatenate([zpre, wk, zpost], axis=3).reshape(C, K))
    w_pack = jnp.stack(slots, axis=0)
    g_col = gamma.reshape(C, 1)
    b_col = beta.reshape(C, 1)

    splits = 2 if groups % 2 == 0 else 1
    per = groups // splits

    gram_p, svec_p = pl.pallas_call(
        functools.partial(_stats_kernel, w_img=w_img, hw=hw),
        out_shape=(
            jax.ShapeDtypeStruct((splits, K, K), jnp.float32),
            jax.ShapeDtypeStruct((splits, K, 1), jnp.float32),
        ),
        grid=(splits, per),
        in_specs=[
            pl.BlockSpec((1, rows, hw), lambda s, t: (s * per + t, 0, 0)),
        ],
        out_specs=(
            pl.BlockSpec((1, K, K), lambda s, t: (s, 0, 0)),
            pl.BlockSpec((1, K, 1), lambda s, t: (s, 0, 0)),
        ),
        compiler_params=pltpu.CompilerParams(
            dimension_semantics=("parallel", "arbitrary")),
    )(xpack)

    g_all = jnp.sum(gram_p, axis=0).reshape(9, pack, cin, 9, pack, cin)
    s_all = jnp.sum(svec_p, axis=0).reshape(9, pack, cin)
    gram = jnp.zeros((9, cin, 9, cin), jnp.float32)
    svec = jnp.zeros((9, cin), jnp.float32)
    for i in range(pack):
        gram = gram + g_all[:, i, :, :, i, :]
        svec = svec + s_all[:, i, :]
    gram = gram.reshape(27, 27)
    svec = svec.reshape(27, 1)

    out = pl.pallas_call(
        functools.partial(_conv_bn_relu_kernel, w_img=w_img, hw=hw,
                          pack=pack, inv_m=1.0 / float(M), eps=eps),
        out_shape=jax.ShapeDtypeStruct((N, C, h_img, w_img), jnp.float32),
        grid=(groups,),
        in_specs=[
            pl.BlockSpec((1, rows, hw), lambda n: (n, 0, 0)),
            pl.BlockSpec((C, 27), lambda n: (0, 0)),
            pl.BlockSpec((pack, C, K), lambda n: (0, 0, 0)),
            pl.BlockSpec((27, 27), lambda n: (0, 0)),
            pl.BlockSpec((27, 1), lambda n: (0, 0)),
            pl.BlockSpec((C, 1), lambda n: (0, 0)),
            pl.BlockSpec((C, 1), lambda n: (0, 0)),
        ],
        out_specs=pl.BlockSpec((pack, C, h_img, w_img),
                               lambda n: (n, 0, 0, 0)),
        compiler_params=pltpu.CompilerParams(
            dimension_semantics=("parallel",)),
    )(xpack, w_mat, w_pack, gram, svec, g_col, b_col)

    return out


# speedup vs baseline: 3.8260x; 3.8260x over previous
import functools

import jax
import jax.numpy as jnp
from jax import lax
from jax.experimental import pallas as pl
from jax.experimental.pallas import tpu as pltpu


def _build_patches(xr, w_img, hw):
    m = lax.broadcasted_iota(jnp.int32, (1, hw), 1)
    w_pos = m % w_img
    col_ok = {
        0: w_pos >= 1,
        1: None,
        2: w_pos <= w_img - 2,
    }
    row_ok = {
        0: m >= w_img,
        1: None,
        2: m < hw - w_img,
    }
    taps = []
    for kh in range(3):
        for kw in range(3):
            d = (kh - 1) * w_img + (kw - 1)
            sh = xr if d == 0 else pltpu.roll(xr, (-d) % hw, axis=1)
            mask = row_ok[kh]
            if col_ok[kw] is not None:
                mask = col_ok[kw] if mask is None else (mask & col_ok[kw])
            if mask is not None:
                sh = jnp.where(mask, sh, 0.0)
            taps.append(sh)
    return jnp.concatenate(taps, axis=0)


def _stats_kernel(x_ref, g_ref, s_ref, *, w_img, hw):
    t = pl.program_id(1)

    @pl.when(t == 0)
    def _():
        g_ref[...] = jnp.zeros_like(g_ref)
        s_ref[...] = jnp.zeros_like(s_ref)

    p = _build_patches(x_ref[0], w_img, hw)
    g_ref[0] += lax.dot_general(p, p, (((1,), (1,)), ((), ())),
                                preferred_element_type=jnp.float32)
    s_ref[0] += jnp.sum(p, axis=1, keepdims=True)


def _conv_bn_relu_kernel(x_ref, wt_ref, wp_ref, g_ref, s_ref, gam_ref,
                         bet_ref, o_ref, *, w_img, hw, pack, inv_m, eps):
    wt = wt_ref[...]
    mean = jnp.dot(s_ref[...], wt,
                   preferred_element_type=jnp.float32) * inv_m
    gw = jnp.dot(g_ref[...], wt, preferred_element_type=jnp.float32)
    sq = jnp.sum(gw * wt, axis=0, keepdims=True)
    var = sq * inv_m - mean * mean
    scale = gam_ref[...] * lax.rsqrt(var + eps)
    shift = bet_ref[...] - mean * scale

    p = _build_patches(x_ref[0], w_img, hw)
    C = wt.shape[1]
    h_img = hw // w_img
    for i in range(pack):
        y = lax.dot_general(p, wp_ref[i], (((0,), (0,)), ((), ())),
                            preferred_element_type=jnp.float32)
        y = jnp.maximum(y * scale + shift, 0.0)
        o_ref[0, i] = y.reshape(h_img, w_img, C)


def kernel(x_nchw, w_hwio, bias, gamma, beta, *, eps=1e-5):
    N, cin, h_img, w_img = x_nchw.shape
    assert cin == 3
    C = w_hwio.shape[-1]
    hw = h_img * w_img
    M = N * hw
    del bias

    pack = 2 if N % 2 == 0 else 1
    groups = N // pack
    rows = cin * pack
    K = 9 * rows

    xpack = x_nchw.reshape(groups, rows, hw)
    w_t = w_hwio.reshape(9 * cin, C)
    wk = w_t.reshape(9, 1, cin, C)
    slots = []
    for i in range(pack):
        zpre = jnp.zeros((9, 1, cin * i, C), jnp.float32)
        zpost = jnp.zeros((9, 1, cin * (pack - 1 - i), C), jnp.float32)
        slots.append(jnp.concatenate([zpre, wk, zpost], axis=2).reshape(K, C))
    w_pack = jnp.stack(slots, axis=0)
    g_row = gamma.reshape(1, C)
    b_row = beta.reshape(1, C)

    splits = 2 if groups % 2 == 0 else 1
    per = groups // splits

    gram_p, svec_p = pl.pallas_call(
        functools.partial(_stats_kernel, w_img=w_img, hw=hw),
        out_shape=(
            jax.ShapeDtypeStruct((splits, K, K), jnp.float32),
            jax.ShapeDtypeStruct((splits, K, 1), jnp.float32),
        ),
        grid=(splits, per),
        in_specs=[
            pl.BlockSpec((1, rows, hw), lambda s, t: (s * per + t, 0, 0)),
        ],
        out_specs=(
            pl.BlockSpec((1, K, K), lambda s, t: (s, 0, 0)),
            pl.BlockSpec((1, K, 1), lambda s, t: (s, 0, 0)),
        ),
        compiler_params=pltpu.CompilerParams(
            dimension_semantics=("parallel", "arbitrary")),
    )(xpack)

    g_all = jnp.sum(gram_p, axis=0).reshape(9, pack, cin, 9, pack, cin)
    s_all = jnp.sum(svec_p, axis=0).reshape(9, pack, cin)
    gram = jnp.zeros((9, cin, 9, cin), jnp.float32)
    svec = jnp.zeros((9, cin), jnp.float32)
    for i in range(pack):
        gram = gram + g_all[:, i, :, :, i, :]
        svec = svec + s_all[:, i, :]
    gram = gram.reshape(27, 27)
    svec = svec.reshape(1, 27)

    out = pl.pallas_call(
        functools.partial(_conv_bn_relu_kernel, w_img=w_img, hw=hw,
                          pack=pack, inv_m=1.0 / float(M), eps=eps),
        out_shape=jax.ShapeDtypeStruct((groups, pack, h_img, w_img, C),
                                       jnp.float32),
        grid=(groups,),
        in_specs=[
            pl.BlockSpec((1, rows, hw), lambda n: (n, 0, 0)),
            pl.BlockSpec((27, C), lambda n: (0, 0)),
            pl.BlockSpec((pack, K, C), lambda n: (0, 0, 0)),
            pl.BlockSpec((27, 27), lambda n: (0, 0)),
            pl.BlockSpec((1, 27), lambda n: (0, 0)),
            pl.BlockSpec((1, C), lambda n: (0, 0)),
            pl.BlockSpec((1, C), lambda n: (0, 0)),
        ],
        out_specs=pl.BlockSpec((1, pack, h_img, w_img, C),
                               lambda n: (n, 0, 0, 0, 0)),
        compiler_params=pltpu.CompilerParams(
            dimension_semantics=("parallel",)),
    )(xpack, w_t, w_pack, gram, svec, g_row, b_row)

    return jnp.transpose(out.reshape(N, h_img, w_img, C), (0, 3, 1, 2))
